# Initial kernel scaffold; baseline (speedup 1.0000x reference)
import jax
import jax.numpy as jnp
from jax.experimental import pallas as pl


def kernel(x, adj_t, W_l1, b_l1, W_r1, gamma1, beta1, W_l2, b_l2, W_r2, gamma2, beta2, W_l3, b_l3, W_r3):
    raise NotImplementedError("write your pallas kernel here")


# speedup vs baseline: 3.1887x; 3.1887x over previous
import functools

import jax
import jax.numpy as jnp
from jax import lax
from jax.experimental import pallas as pl
from jax.experimental.pallas import tpu as pltpu
from jax.experimental.pallas import tpu_sc as plsc

NN = 10000
NP = 10240
EE = 160000
EP = 163840
D = 256
DH = 128
BN_ROWS = 2048
GRID = NP // BN_ROWS
W_SEG = 80
W_CNT = 40


def _seg_mesh():
    return plsc.VectorSubcoreMesh(core_axis_name="c", subcore_axis_name="s")


def _count_body(dst_hbm, zeros_hbm, ones_hbm, cnt_hbm, didx, ones_v, acc):
    c = lax.axis_index("c")
    s = lax.axis_index("s")
    pltpu.sync_copy(zeros_hbm, acc.at[pl.ds(s * 640, 640)])
    pltpu.sync_copy(ones_hbm, ones_v)
    pltpu.sync_copy(dst_hbm.at[c, s], didx)
    plsc.subcore_barrier()

    def body(j, carry):
        pltpu.sync_copy(ones_v, acc.at[didx.at[j]], add=True)
        return carry

    lax.fori_loop(0, W_CNT, body, 0)
    plsc.subcore_barrier()
    pltpu.sync_copy(acc.at[pl.ds(s * 640, 640)],
                    cnt_hbm.at[c, pl.ds(s * 640, 640)])


def _sc_count(dst_cnt, zeros_cnt, ones128):
    return pl.kernel(
        _count_body,
        out_type=jax.ShapeDtypeStruct((2, NP), jnp.float32),
        mesh=_seg_mesh(),
        scratch_types=[
            pltpu.VMEM((W_CNT, 128), jnp.int32),
            pltpu.VMEM((128,), jnp.float32),
            pltpu.VMEM_SHARED((NP,), jnp.float32),
        ],
    )(dst_cnt, zeros_cnt, ones128)


def _segsum_body(tab_hbm, src_hbm, dst_hbm, zeros_hbm, agg_hbm,
                 sidx, didx, rows, acc):
    c = lax.axis_index("c")
    s = lax.axis_index("s")
    pltpu.sync_copy(zeros_hbm, acc.at[pl.ds(s * 640, 640)])
    pltpu.sync_copy(src_hbm.at[c, s], sidx)
    pltpu.sync_copy(dst_hbm.at[s], didx)
    plsc.subcore_barrier()

    def body(j, carry):
        pltpu.sync_copy(tab_hbm.at[sidx.at[j]], rows)
        pltpu.sync_copy(rows, acc.at[didx.at[j]], add=True)
        return carry

    lax.fori_loop(0, W_SEG, body, 0)
    plsc.subcore_barrier()
    pltpu.sync_copy(acc.at[pl.ds(s * 640, 640)],
                    agg_hbm.at[c, pl.ds(s * 640, 640)])


def _sc_segsum(tab_flat, srcs, dsts, zeros_row):
    return pl.kernel(
        _segsum_body,
        out_type=jax.ShapeDtypeStruct((2, NP, DH), jnp.float32),
        mesh=_seg_mesh(),
        scratch_types=[
            pltpu.VMEM((W_SEG, 128), jnp.int32),
            pltpu.VMEM((W_SEG, 128), jnp.int32),
            pltpu.VMEM((128, DH), jnp.float32),
            pltpu.VMEM_SHARED((NP, DH), jnp.float32),
        ],
    )(tab_flat, srcs, dsts, zeros_row)


def _linear_body(do_stats, agg_ref, cnt_ref, tab_ref, w_ref, b_ref,
                 h_ref, *rest):
    if do_stats:
        stats_ref, sacc = rest
    else:
        (sacc,) = rest
    i = pl.program_id(0)
    cnt = cnt_ref[0] + cnt_ref[1]
    inv = 1.0 / jnp.clip(cnt, 1.0, None)
    mean = jnp.concatenate([agg_ref[0], agg_ref[1]], axis=1) * inv
    xb = jnp.concatenate([tab_ref[0], tab_ref[1]], axis=1)
    hcat = jnp.concatenate([mean, xb], axis=1)
    h = jnp.dot(hcat, w_ref[...], preferred_element_type=jnp.float32)
    h = h + b_ref[...]
    h_ref[...] = h
    if do_stats:
        rows = i * BN_ROWS + lax.broadcasted_iota(jnp.int32, (BN_ROWS, D), 0)
        hm = jnp.where(rows < NN, h, 0.0)

        @pl.when(i == 0)
        def _():
            sacc[...] = jnp.zeros_like(sacc)

        sacc[0:1, :] += jnp.sum(hm, axis=0, keepdims=True)
        sacc[1:2, :] += jnp.sum(hm * hm, axis=0, keepdims=True)

        @pl.when(i == GRID - 1)
        def _():
            mu = sacc[0:1, :] / float(NN)
            stats_ref[0:1, :] = mu
            stats_ref[1:2, :] = sacc[1:2, :] / float(NN) - mu * mu


def _tc_linear(agg, cnt3, tab, w_cat, b_row, do_stats):
    out_shape = [jax.ShapeDtypeStruct((NP, D), jnp.float32)]
    out_specs = [pl.BlockSpec((BN_ROWS, D), lambda i: (i, 0))]
    if do_stats:
        out_shape.append(jax.ShapeDtypeStruct((2, D), jnp.float32))
        out_specs.append(pl.BlockSpec((2, D), lambda i: (0, 0)))
    res = pl.pallas_call(
        functools.partial(_linear_body, do_stats),
        grid=(GRID,),
        in_specs=[
            pl.BlockSpec((2, BN_ROWS, DH), lambda i: (0, i, 0)),
            pl.BlockSpec((2, BN_ROWS, 1), lambda i: (0, i, 0)),
            pl.BlockSpec((2, BN_ROWS, DH), lambda i: (0, i, 0)),
            pl.BlockSpec((2 * D, D), lambda i: (0, 0)),
            pl.BlockSpec((1, D), lambda i: (0, 0)),
        ],
        out_specs=out_specs if do_stats else out_specs[0],
        out_shape=out_shape if do_stats else out_shape[0],
        scratch_shapes=[pltpu.VMEM((2, D), jnp.float32)],
    )(agg, cnt3, tab, w_cat, b_row)
    return res if do_stats else (res, None)


def _bn_relu_body(h_ref, stats_ref, gam_ref, bet_ref, out_ref):
    h = h_ref[...]
    mu = stats_ref[0:1, :]
    var = stats_ref[1:2, :]
    y = gam_ref[...] * (h - mu) * lax.rsqrt(var + 1e-5) + bet_ref[...]
    y = jnp.maximum(y, 0.0)
    out_ref[0] = y[:, :DH]
    out_ref[1] = y[:, DH:]


def _tc_bn_relu(h, stats, gamma, beta):
    return pl.pallas_call(
        _bn_relu_body,
        grid=(GRID,),
        in_specs=[
            pl.BlockSpec((BN_ROWS, D), lambda i: (i, 0)),
            pl.BlockSpec((2, D), lambda i: (0, 0)),
            pl.BlockSpec((1, D), lambda i: (0, 0)),
            pl.BlockSpec((1, D), lambda i: (0, 0)),
        ],
        out_specs=pl.BlockSpec((2, BN_ROWS, DH), lambda i: (0, i, 0)),
        out_shape=jax.ShapeDtypeStruct((2, NP, DH), jnp.float32),
    )(h, stats, gamma.reshape(1, D), beta.reshape(1, D))


def kernel(x, adj_t, W_l1, b_l1, W_r1, gamma1, beta1,
           W_l2, b_l2, W_r2, gamma2, beta2, W_l3, b_l3, W_r3):
    src = adj_t[0]
    dst = adj_t[1]
    pad = EP - EE
    src_p = jnp.concatenate([src, jnp.zeros((pad,), jnp.int32)])
    dst_p = jnp.concatenate(
        [dst, NN + (jnp.arange(pad, dtype=jnp.int32) % (NP - NN))])
    srcs = jnp.stack([src_p, src_p + NP]).reshape(2, 16, W_SEG, 128)
    dsts_seg = dst_p.reshape(16, W_SEG, 128)
    dsts_cnt = dst_p.reshape(2, 16, W_CNT, 128)

    zeros_row = jnp.zeros((640, DH), jnp.float32)
    zeros_cnt = jnp.zeros((640,), jnp.float32)
    ones128 = jnp.ones((128,), jnp.float32)

    cnt = _sc_count(dsts_cnt, zeros_cnt, ones128)
    cnt3 = cnt.reshape(2, NP, 1)

    xs = jnp.pad(x, ((0, NP - NN), (0, 0))).reshape(NP, 2, DH)
    xs = xs.transpose(1, 0, 2)

    wc1 = jnp.concatenate([W_l1.T, W_r1.T], axis=0)
    wc2 = jnp.concatenate([W_l2.T, W_r2.T], axis=0)
    wc3 = jnp.concatenate([W_l3.T, W_r3.T], axis=0)

    agg1 = _sc_segsum(xs.reshape(2 * NP, DH), srcs, dsts_seg, zeros_row)
    h1, st1 = _tc_linear(agg1, cnt3, xs, wc1, b_l1.reshape(1, D), True)
    t2 = _tc_bn_relu(h1, st1, gamma1, beta1)

    agg2 = _sc_segsum(t2.reshape(2 * NP, DH), srcs, dsts_seg, zeros_row)
    h2, st2 = _tc_linear(agg2, cnt3, t2, wc2, b_l2.reshape(1, D), True)
    t3 = _tc_bn_relu(h2, st2, gamma2, beta2)

    agg3 = _sc_segsum(t3.reshape(2 * NP, DH), srcs, dsts_seg, zeros_row)
    out_p, _ = _tc_linear(agg3, cnt3, t3, wc3, b_l3.reshape(1, D), False)
    return out_p[:NN]


# speedup vs baseline: 3.5537x; 1.1145x over previous
import functools

import jax
import jax.numpy as jnp
from jax import lax
from jax.experimental import pallas as pl
from jax.experimental.pallas import tpu as pltpu
from jax.experimental.pallas import tpu_sc as plsc

NN = 10000
NP = 10240
ACCN = 10240
EE = 160000
EP = 163840
D = 256
DH = 128
BN_ROWS = 2048
GRID = NP // BN_ROWS
W_SEG = 80
W_EDGE = 128
W_HALF = 2
W_CHUNK = W_SEG // W_HALF
W_CNT = 40


def _seg_mesh():
    return plsc.VectorSubcoreMesh(core_axis_name="c", subcore_axis_name="s")


def _count_body(dst_hbm, zeros_hbm, ones_hbm, cnt_hbm, didx, ones_v, acc):
    c = lax.axis_index("c")
    s = lax.axis_index("s")
    pltpu.sync_copy(zeros_hbm, acc.at[pl.ds(s * 640, 640)])
    pltpu.sync_copy(ones_hbm, ones_v)
    pltpu.sync_copy(dst_hbm.at[c, s], didx)
    plsc.subcore_barrier()

    def body(j, carry):
        pltpu.sync_copy(ones_v, acc.at[didx.at[j]], add=True)
        return carry

    lax.fori_loop(0, W_CNT, body, 0)
    plsc.subcore_barrier()
    pltpu.sync_copy(acc.at[pl.ds(s * 640, 640)],
                    cnt_hbm.at[c, pl.ds(s * 640, 640)])


def _sc_count(dst_cnt, zeros_cnt, ones128):
    return pl.kernel(
        _count_body,
        out_type=jax.ShapeDtypeStruct((2, NP), jnp.float32),
        mesh=_seg_mesh(),
        scratch_types=[
            pltpu.VMEM((W_CNT, 128), jnp.int32),
            pltpu.VMEM((128,), jnp.float32),
            pltpu.VMEM_SHARED((NP,), jnp.float32),
        ],
    )(dst_cnt, zeros_cnt, ones128)


def _segsum_body(tab_hbm, src_hbm, dst_hbm, zeros_hbm, agg_hbm,
                 sidx, didx, rows, acc, sem_g, sem_s):
    c = lax.axis_index("c")
    s = lax.axis_index("s")
    pltpu.sync_copy(zeros_hbm, acc.at[pl.ds(s * 640, 640)])
    plsc.subcore_barrier()

    for h in range(W_HALF):
        pltpu.sync_copy(src_hbm.at[c, s, pl.ds(h * W_CHUNK, W_CHUNK)], sidx)
        pltpu.sync_copy(dst_hbm.at[s, pl.ds(h * W_CHUNK, W_CHUNK)], didx)
        pltpu.async_copy(tab_hbm.at[sidx.at[0]], rows.at[0], sem_g.at[0])

        def body(j, carry):
            p = j % 2
            pltpu.make_async_copy(tab_hbm.at[sidx.at[j]], rows.at[p],
                                  sem_g.at[p]).wait()
            pltpu.async_copy(rows.at[p], acc.at[didx.at[j]], sem_s.at[p],
                             add=True)

            @pl.when(j + 1 < W_CHUNK)
            def _():
                pn = (j + 1) % 2

                @pl.when(j >= 1)
                def _():
                    pltpu.make_async_copy(rows.at[pn],
                                          acc.at[didx.at[j - 1]],
                                          sem_s.at[pn]).wait()

                pltpu.async_copy(tab_hbm.at[sidx.at[j + 1]], rows.at[pn],
                                 sem_g.at[pn])

            return carry

        lax.fori_loop(0, W_CHUNK, body, 0)
        pltpu.make_async_copy(rows.at[(W_CHUNK - 2) % 2],
                              acc.at[didx.at[W_CHUNK - 2]],
                              sem_s.at[(W_CHUNK - 2) % 2]).wait()
        pltpu.make_async_copy(rows.at[(W_CHUNK - 1) % 2],
                              acc.at[didx.at[W_CHUNK - 1]],
                              sem_s.at[(W_CHUNK - 1) % 2]).wait()
    plsc.subcore_barrier()
    pltpu.sync_copy(acc.at[pl.ds(s * 640, 640)],
                    agg_hbm.at[c, pl.ds(s * 640, 640)])


def _sc_segsum(tab_flat, srcs, dsts, zeros_row):
    return pl.kernel(
        _segsum_body,
        out_type=jax.ShapeDtypeStruct((2, NP, DH), jnp.float32),
        mesh=_seg_mesh(),
        scratch_types=[
            pltpu.VMEM((W_CHUNK, W_EDGE), jnp.int32),
            pltpu.VMEM((W_CHUNK, W_EDGE), jnp.int32),
            pltpu.VMEM((2, W_EDGE, DH), jnp.float32),
            pltpu.VMEM_SHARED((ACCN, DH), jnp.float32),
            pltpu.SemaphoreType.DMA((2,)),
            pltpu.SemaphoreType.DMA((2,)),
        ],
    )(tab_flat, srcs, dsts, zeros_row)


def _linear_body(do_stats, agg_ref, cnt_ref, tab_ref, w_ref, b_ref,
                 h_ref, *rest):
    if do_stats:
        stats_ref, sacc = rest
    else:
        (sacc,) = rest
    i = pl.program_id(0)
    cnt = cnt_ref[0] + cnt_ref[1]
    inv = 1.0 / jnp.clip(cnt, 1.0, None)
    mean = jnp.concatenate([agg_ref[0], agg_ref[1]], axis=1) * inv
    xb = jnp.concatenate([tab_ref[0], tab_ref[1]], axis=1)
    hcat = jnp.concatenate([mean, xb], axis=1)
    h = jnp.dot(hcat, w_ref[...], preferred_element_type=jnp.float32)
    h = h + b_ref[...]
    h_ref[...] = h
    if do_stats:
        rows = i * BN_ROWS + lax.broadcasted_iota(jnp.int32, (BN_ROWS, D), 0)
        hm = jnp.where(rows < NN, h, 0.0)

        @pl.when(i == 0)
        def _():
            sacc[...] = jnp.zeros_like(sacc)

        sacc[0:1, :] += jnp.sum(hm, axis=0, keepdims=True)
        sacc[1:2, :] += jnp.sum(hm * hm, axis=0, keepdims=True)

        @pl.when(i == GRID - 1)
        def _():
            mu = sacc[0:1, :] / float(NN)
            stats_ref[0:1, :] = mu
            stats_ref[1:2, :] = sacc[1:2, :] / float(NN) - mu * mu


def _tc_linear(agg, cnt3, tab, w_cat, b_row, do_stats):
    out_shape = [jax.ShapeDtypeStruct((NP, D), jnp.float32)]
    out_specs = [pl.BlockSpec((BN_ROWS, D), lambda i: (i, 0))]
    if do_stats:
        out_shape.append(jax.ShapeDtypeStruct((2, D), jnp.float32))
        out_specs.append(pl.BlockSpec((2, D), lambda i: (0, 0)))
    res = pl.pallas_call(
        functools.partial(_linear_body, do_stats),
        grid=(GRID,),
        in_specs=[
            pl.BlockSpec((2, BN_ROWS, DH), lambda i: (0, i, 0)),
            pl.BlockSpec((2, BN_ROWS, 1), lambda i: (0, i, 0)),
            pl.BlockSpec((2, BN_ROWS, DH), lambda i: (0, i, 0)),
            pl.BlockSpec((2 * D, D), lambda i: (0, 0)),
            pl.BlockSpec((1, D), lambda i: (0, 0)),
        ],
        out_specs=out_specs if do_stats else out_specs[0],
        out_shape=out_shape if do_stats else out_shape[0],
        scratch_shapes=[pltpu.VMEM((2, D), jnp.float32)],
    )(agg, cnt3, tab, w_cat, b_row)
    return res if do_stats else (res, None)


def _bn_relu_body(h_ref, stats_ref, gam_ref, bet_ref, out_ref):
    h = h_ref[...]
    mu = stats_ref[0:1, :]
    var = stats_ref[1:2, :]
    y = gam_ref[...] * (h - mu) * lax.rsqrt(var + 1e-5) + bet_ref[...]
    y = jnp.maximum(y, 0.0)
    out_ref[0] = y[:, :DH]
    out_ref[1] = y[:, DH:]


def _tc_bn_relu(h, stats, gamma, beta):
    return pl.pallas_call(
        _bn_relu_body,
        grid=(GRID,),
        in_specs=[
            pl.BlockSpec((BN_ROWS, D), lambda i: (i, 0)),
            pl.BlockSpec((2, D), lambda i: (0, 0)),
            pl.BlockSpec((1, D), lambda i: (0, 0)),
            pl.BlockSpec((1, D), lambda i: (0, 0)),
        ],
        out_specs=pl.BlockSpec((2, BN_ROWS, DH), lambda i: (0, i, 0)),
        out_shape=jax.ShapeDtypeStruct((2, NP, DH), jnp.float32),
    )(h, stats, gamma.reshape(1, D), beta.reshape(1, D))


def kernel(x, adj_t, W_l1, b_l1, W_r1, gamma1, beta1,
           W_l2, b_l2, W_r2, gamma2, beta2, W_l3, b_l3, W_r3):
    src = adj_t[0]
    dst = adj_t[1]
    pad = EP - EE
    src_p = jnp.concatenate([src, jnp.zeros((pad,), jnp.int32)])
    dst_p = jnp.concatenate(
        [dst, NN + (jnp.arange(pad, dtype=jnp.int32) % (ACCN - NN))])
    srcs = jnp.stack([src_p, src_p + NP]).reshape(2, 16, W_SEG, W_EDGE)
    dsts_seg = dst_p.reshape(16, W_SEG, W_EDGE)
    dsts_cnt = dst_p.reshape(2, 16, W_CNT, 128)

    zeros_row = jnp.zeros((640, DH), jnp.float32)
    zeros_cnt = jnp.zeros((640,), jnp.float32)
    ones128 = jnp.ones((128,), jnp.float32)

    cnt = _sc_count(dsts_cnt, zeros_cnt, ones128)
    cnt3 = cnt.reshape(2, NP, 1)

    xs = jnp.pad(x, ((0, NP - NN), (0, 0))).reshape(NP, 2, DH)
    xs = xs.transpose(1, 0, 2)

    wc1 = jnp.concatenate([W_l1.T, W_r1.T], axis=0)
    wc2 = jnp.concatenate([W_l2.T, W_r2.T], axis=0)
    wc3 = jnp.concatenate([W_l3.T, W_r3.T], axis=0)

    agg1 = _sc_segsum(xs.reshape(2 * NP, DH), srcs, dsts_seg, zeros_row)
    h1, st1 = _tc_linear(agg1, cnt3, xs, wc1, b_l1.reshape(1, D), True)
    t2 = _tc_bn_relu(h1, st1, gamma1, beta1)

    agg2 = _sc_segsum(t2.reshape(2 * NP, DH), srcs, dsts_seg, zeros_row)
    h2, st2 = _tc_linear(agg2, cnt3, t2, wc2, b_l2.reshape(1, D), True)
    t3 = _tc_bn_relu(h2, st2, gamma2, beta2)

    agg3 = _sc_segsum(t3.reshape(2 * NP, DH), srcs, dsts_seg, zeros_row)
    out_p, _ = _tc_linear(agg3, cnt3, t3, wc3, b_l3.reshape(1, D), False)
    return out_p[:NN]
